# baseline (speedup 1.0000x reference)
import functools

import jax
import jax.numpy as jnp
import numpy as np
from jax import lax
from jax.experimental import pallas as pl
from jax.experimental.pallas import tpu as pltpu
from jax.experimental.pallas import tpu_sc as plsc

B = 1024
V = 100000
K = 20
NW = 32
ROWS_PER_W = B // NW
ELEMS_PER_W = ROWS_PER_W * K
CHUNK = 128
N_CHUNKS = ELEMS_PER_W // CHUNK
N_VECS = ELEMS_PER_W // 16
LN2 = 0.6931471805599453
SCALE = 1.0 / (B * K)


def _ln(x):
    xi = lax.bitcast_convert_type(x, jnp.int32)
    e = (xi >> 23) - 127
    m = lax.bitcast_convert_type((xi & 0x007FFFFF) | 0x3F800000, jnp.float32)
    s = (m - 1.0) / (m + 1.0)
    s2 = s * s
    poly = 1.0 + s2 * (1.0 / 3 + s2 * (1.0 / 5 + s2 * (1.0 / 7 + s2 * (1.0 / 9))))
    return e.astype(jnp.float32) * LN2 + 2.0 * s * poly


def _sc_body(table, ko_ids, en_ids, out, ko_idx, en_idx, ko_vals, en_vals,
             out_v, sem):
    wid = lax.axis_index("s") * 2 + lax.axis_index("c")
    base_el = wid * ELEMS_PER_W
    base_row = wid * ROWS_PER_W

    pltpu.sync_copy(ko_ids.at[pl.ds(base_el, ELEMS_PER_W)], ko_idx)
    pltpu.sync_copy(en_ids.at[pl.ds(base_el, ELEMS_PER_W)], en_idx)

    lane = lax.iota(jnp.int32, 16)
    base_off = base_row * V
    for i in range(N_VECS):
        r0 = (i * 16) // K
        cut = (r0 + 1) * K - i * 16
        if cut >= 16:
            off = base_off + r0 * V
        else:
            off = jnp.where(lane >= cut, base_off + (r0 + 1) * V,
                            base_off + r0 * V)
        ko_idx[pl.ds(i * 16, 16)] = ko_idx[pl.ds(i * 16, 16)] + off
        en_idx[pl.ds(i * 16, 16)] = en_idx[pl.ds(i * 16, 16)] + off

    copies = []
    for c in range(N_CHUNKS):
        copies.append(pltpu.async_copy(
            table.at[ko_idx.at[pl.ds(c * CHUNK, CHUNK)]],
            ko_vals.at[pl.ds(c * CHUNK, CHUNK)], sem))
        copies.append(pltpu.async_copy(
            table.at[en_idx.at[pl.ds(c * CHUNK, CHUNK)]],
            en_vals.at[pl.ds(c * CHUNK, CHUNK)], sem))
    for cp in copies:
        cp.wait()

    a_ko = jnp.zeros((16,), jnp.float32)
    a_en = jnp.zeros((16,), jnp.float32)
    a_rl = jnp.zeros((16,), jnp.float32)
    for i in range(N_VECS):
        vko = ko_vals[pl.ds(i * 16, 16)]
        ven = en_vals[pl.ds(i * 16, 16)]
        a_ko = a_ko + _ln(vko + 1e-8)
        a_en = a_en + _ln(ven + 1e-8)
        a_rl = a_rl + jnp.maximum(2.0 - ven, 0.0)

    out_v[pl.ds(0, 16)] = a_ko
    out_v[pl.ds(16, 16)] = a_en
    out_v[pl.ds(32, 16)] = a_rl
    pltpu.sync_copy(out_v, out.at[pl.ds(wid * 48, 48)])


_sc_call = functools.partial(
    pl.kernel,
    out_type=jax.ShapeDtypeStruct((NW * 48,), jnp.float32),
    mesh=plsc.VectorSubcoreMesh(core_axis_name="c", subcore_axis_name="s"),
    scratch_types=[
        pltpu.VMEM((ELEMS_PER_W,), jnp.int32),
        pltpu.VMEM((ELEMS_PER_W,), jnp.int32),
        pltpu.VMEM((ELEMS_PER_W,), jnp.float32),
        pltpu.VMEM((ELEMS_PER_W,), jnp.float32),
        pltpu.VMEM((48,), jnp.float32),
        pltpu.SemaphoreType.DMA,
    ],
)(_sc_body)


def kernel(sparse_rep, ko_token_ids, en_token_ids):
    partials = _sc_call(sparse_rep.reshape(-1),
                        ko_token_ids.reshape(-1),
                        en_token_ids.reshape(-1))
    sums = partials.reshape(NW, 3, 16).sum(axis=(0, 2))
    return jnp.stack([-sums[0] * SCALE, -sums[1] * SCALE, sums[2] * SCALE,
                      jnp.zeros((), jnp.float32)])
